# Initial kernel scaffold; baseline (speedup 1.0000x reference)
import functools

import jax
import jax.numpy as jnp
from jax import lax
from jax.experimental import pallas as pl
from jax.experimental.pallas import tpu as pltpu
from jax.experimental.pallas import tpu_sc as plsc

B, N, F_IN, H, D_OUT, E = 4, 10000, 128, 64, 64, 320000
NC, NS = 2, 16
NW = NC * NS
EP = E // NW
K = 80
NCH = EP // K
RPT = N // NS
FH = B * H // 2

_mesh = plsc.VectorSubcoreMesh(
    core_axis_name="c", subcore_axis_name="s", num_cores=NC, num_subcores=NS)


@functools.partial(
    pl.kernel,
    out_type=jax.ShapeDtypeStruct((NC * N, 16), jnp.float32),
    mesh=_mesh,
    scratch_types=[
        pltpu.VMEM((NCH, K), jnp.int32),
        pltpu.VMEM((RPT, 16), jnp.float32),
        pltpu.VMEM((K, 16), jnp.float32),
        pltpu.VMEM_SHARED((N, 16), jnp.float32),
    ],
)
def _sc_deg(dst_h, zeros_h, ones_h, out, dstv, zbuf, onesb, acc):
    c = lax.axis_index("c")
    s = lax.axis_index("s")
    w = c * NS + s
    pltpu.sync_copy(dst_h.at[pl.ds(w * NCH, NCH)], dstv)
    pltpu.sync_copy(zeros_h, zbuf)
    pltpu.sync_copy(ones_h, onesb)
    pltpu.sync_copy(zbuf, acc.at[pl.ds(s * RPT, RPT)])
    plsc.subcore_barrier()

    def chunk(j, carry):
        pltpu.sync_copy(onesb, acc.at[dstv.at[j]], add=True)
        return carry

    lax.fori_loop(0, NCH, chunk, 0)
    plsc.subcore_barrier()
    pltpu.sync_copy(acc.at[pl.ds(s * RPT, RPT)], zbuf)
    pltpu.sync_copy(zbuf, out.at[pl.ds(c * N + s * RPT, RPT)])


@functools.partial(
    pl.kernel,
    out_type=jax.ShapeDtypeStruct((NC * N, FH), jnp.float32),
    mesh=_mesh,
    scratch_types=[
        pltpu.VMEM((NCH, K), jnp.int32),
        pltpu.VMEM((NCH, K), jnp.int32),
        pltpu.VMEM((K, FH), jnp.float32),
        pltpu.VMEM((NCH, FH), jnp.float32),
        pltpu.VMEM_SHARED((N, FH), jnp.float32),
        pltpu.SemaphoreType.DMA,
    ],
)
def _sc_agg(xs0, xs1, src_h, dst_h, zeros_h, out, srcv, dstv, rows, zbuf,
            acc, sem):
    c = lax.axis_index("c")
    s = lax.axis_index("s")
    w = c * NS + s
    pltpu.sync_copy(src_h.at[pl.ds(w * NCH, NCH)], srcv)
    pltpu.sync_copy(dst_h.at[pl.ds(w * NCH, NCH)], dstv)
    pltpu.sync_copy(zeros_h, zbuf)
    for j in range(RPT // NCH):
        pltpu.sync_copy(zbuf, acc.at[pl.ds(s * RPT + j * NCH, NCH)])
    plsc.subcore_barrier()

    def chunk(j, carry):
        @pl.when(c == 0)
        def _():
            pltpu.async_copy(xs0.at[srcv.at[j]], rows, sem).wait()

        @pl.when(c == 1)
        def _():
            pltpu.async_copy(xs1.at[srcv.at[j]], rows, sem).wait()

        pltpu.sync_copy(rows, acc.at[dstv.at[j]], add=True)
        return carry

    lax.fori_loop(0, NCH, chunk, 0)
    plsc.subcore_barrier()
    for j in range(RPT // NCH):
        pltpu.sync_copy(acc.at[pl.ds(s * RPT + j * NCH, NCH)], zbuf)
        pltpu.sync_copy(
            zbuf, out.at[pl.ds(c * N + s * RPT + j * NCH, NCH)])


BN = 1000
NBLK = N // BN


def _tc_prep_body(x_ref, w_ref, dega_ref, degb_ref, xs0_ref, xs1_ref,
                  dinv_ref):
    deg = dega_ref[:, 0] + degb_ref[:, 0] + 1.0
    dinv = lax.rsqrt(deg)[:, None]
    w = w_ref[...]
    xw = [jnp.dot(x_ref[b], w, preferred_element_type=jnp.float32) * dinv
          for b in range(B)]
    xs0_ref[...] = jnp.concatenate([xw[0], xw[1]], axis=1)
    xs1_ref[...] = jnp.concatenate([xw[2], xw[3]], axis=1)
    dinv_ref[...] = dinv


def _tc_prep(x, w1t, degs):
    return pl.pallas_call(
        _tc_prep_body,
        grid=(NBLK,),
        in_specs=[
            pl.BlockSpec((B, BN, F_IN), lambda i: (0, i, 0)),
            pl.BlockSpec((F_IN, H), lambda i: (0, 0)),
            pl.BlockSpec((BN, 16), lambda i: (i, 0)),
            pl.BlockSpec((BN, 16), lambda i: (NBLK + i, 0)),
        ],
        out_specs=[
            pl.BlockSpec((BN, FH), lambda i: (i, 0)),
            pl.BlockSpec((BN, FH), lambda i: (i, 0)),
            pl.BlockSpec((BN, 1), lambda i: (i, 0)),
        ],
        out_shape=[
            jax.ShapeDtypeStruct((N, FH), jnp.float32),
            jax.ShapeDtypeStruct((N, FH), jnp.float32),
            jax.ShapeDtypeStruct((N, 1), jnp.float32),
        ],
    )(x, w1t, degs, degs)


def _ln_relu(seg, g, be):
    mu = jnp.mean(seg, axis=1, keepdims=True)
    d = seg - mu
    var = jnp.mean(d * d, axis=1, keepdims=True)
    return jnp.maximum(d * lax.rsqrt(var + 1e-5) * g + be, 0.0)


def _tc_mid_body(agga_ref, aggb_ref, xs0_ref, xs1_ref, dinv_ref, b1_ref,
                 g1_ref, be1_ref, w2_ref, o0_ref, o1_ref):
    dinv = dinv_ref[...]
    b1, g1, be1 = b1_ref[...], g1_ref[...], be1_ref[...]
    w2 = w2_ref[...]
    for agg_ref, xs_ref, o_ref in [(agga_ref, xs0_ref, o0_ref),
                                   (aggb_ref, xs1_ref, o1_ref)]:
        outs = []
        for j in range(2):
            sl = slice(j * H, (j + 1) * H)
            pre = dinv * (agg_ref[:, sl] + xs_ref[:, sl]) + b1
            x2 = _ln_relu(pre, g1, be1)
            outs.append(
                jnp.dot(x2, w2, preferred_element_type=jnp.float32) * dinv)
        o_ref[...] = jnp.concatenate(outs, axis=1)


def _tc_mid(agg1, xs0, xs1, dinv, b1, g1, be1, w2t):
    return pl.pallas_call(
        _tc_mid_body,
        grid=(NBLK,),
        in_specs=[
            pl.BlockSpec((BN, FH), lambda i: (i, 0)),
            pl.BlockSpec((BN, FH), lambda i: (NBLK + i, 0)),
            pl.BlockSpec((BN, FH), lambda i: (i, 0)),
            pl.BlockSpec((BN, FH), lambda i: (i, 0)),
            pl.BlockSpec((BN, 1), lambda i: (i, 0)),
            pl.BlockSpec((1, H), lambda i: (0, 0)),
            pl.BlockSpec((1, H), lambda i: (0, 0)),
            pl.BlockSpec((1, H), lambda i: (0, 0)),
            pl.BlockSpec((H, D_OUT), lambda i: (0, 0)),
        ],
        out_specs=[
            pl.BlockSpec((BN, FH), lambda i: (i, 0)),
            pl.BlockSpec((BN, FH), lambda i: (i, 0)),
        ],
        out_shape=[
            jax.ShapeDtypeStruct((N, FH), jnp.float32),
            jax.ShapeDtypeStruct((N, FH), jnp.float32),
        ],
    )(agg1, agg1, xs0, xs1, dinv, b1, g1, be1, w2t)


def _tc_post_body(agga_ref, aggb_ref, xs0_ref, xs1_ref, dinv_ref, b2_ref,
                  g2_ref, be2_ref, embs_ref, graph_ref):
    i = pl.program_id(0)
    dinv = dinv_ref[...]
    b2, g2, be2 = b2_ref[...], g2_ref[...], be2_ref[...]
    ys = []
    for agg_ref, xs_ref in [(agga_ref, xs0_ref), (aggb_ref, xs1_ref)]:
        for j in range(2):
            sl = slice(j * D_OUT, (j + 1) * D_OUT)
            pre = dinv * (agg_ref[:, sl] + xs_ref[:, sl]) + b2
            ys.append(_ln_relu(pre, g2, be2))
    stacked = jnp.stack(ys, axis=0)
    embs_ref[...] = stacked
    part = jnp.sum(stacked, axis=1)

    @pl.when(i == 0)
    def _():
        graph_ref[...] = part

    @pl.when(i > 0)
    def _():
        graph_ref[...] = graph_ref[...] + part

    @pl.when(i == NBLK - 1)
    def _():
        graph_ref[...] = graph_ref[...] * (1.0 / N)


def _tc_post(agg2, xs2_0, xs2_1, dinv, b2, g2, be2):
    return pl.pallas_call(
        _tc_post_body,
        grid=(NBLK,),
        in_specs=[
            pl.BlockSpec((BN, FH), lambda i: (i, 0)),
            pl.BlockSpec((BN, FH), lambda i: (NBLK + i, 0)),
            pl.BlockSpec((BN, FH), lambda i: (i, 0)),
            pl.BlockSpec((BN, FH), lambda i: (i, 0)),
            pl.BlockSpec((BN, 1), lambda i: (i, 0)),
            pl.BlockSpec((1, D_OUT), lambda i: (0, 0)),
            pl.BlockSpec((1, D_OUT), lambda i: (0, 0)),
            pl.BlockSpec((1, D_OUT), lambda i: (0, 0)),
        ],
        out_specs=[
            pl.BlockSpec((B, BN, D_OUT), lambda i: (0, i, 0)),
            pl.BlockSpec((B, D_OUT), lambda i: (0, 0)),
        ],
        out_shape=[
            jax.ShapeDtypeStruct((B, N, D_OUT), jnp.float32),
            jax.ShapeDtypeStruct((B, D_OUT), jnp.float32),
        ],
    )(agg2, agg2, xs2_0, xs2_1, dinv, b2, g2, be2)


def kernel(node_features, edge_index, W1, b1, g1, be1, W2, b2, g2, be2):
    src = edge_index[0].reshape(E // K, K)
    dst = edge_index[1].reshape(E // K, K)
    zeros_fh = jnp.zeros((NCH, FH), jnp.float32)
    zeros16 = jnp.zeros((RPT, 16), jnp.float32)
    ones16 = jnp.ones((K, 16), jnp.float32)

    degs = _sc_deg(dst, zeros16, ones16)
    xs0, xs1, dinv = _tc_prep(node_features, W1.T, degs)
    agg1 = _sc_agg(xs0, xs1, src, dst, zeros_fh)
    xs2_0, xs2_1 = _tc_mid(agg1, xs0, xs1, dinv,
                           b1.reshape(1, H), g1.reshape(1, H),
                           be1.reshape(1, H), W2.T)
    agg2 = _sc_agg(xs2_0, xs2_1, src, dst, zeros_fh)
    node_embs, graph_emb = _tc_post(agg2, xs2_0, xs2_1, dinv,
                                    b2.reshape(1, D_OUT),
                                    g2.reshape(1, D_OUT),
                                    be2.reshape(1, D_OUT))
    return (node_embs, graph_emb)


# speedup vs baseline: 48.0773x; 48.0773x over previous
import functools

import jax
import jax.numpy as jnp
from jax import lax
from jax.experimental import pallas as pl
from jax.experimental.pallas import tpu as pltpu
from jax.experimental.pallas import tpu_sc as plsc

B, N, F_IN, H, D_OUT, E = 4, 10000, 128, 64, 64, 320000
NC, NS = 2, 16
NW = NC * NS
K = 80
NCHD = (E // NW) // K
NCHA = (E // NS) // K
BI = 10
NBI = NCHA // BI
BID = 25
NBID = NCHD // BID
ZR = 40
NZC = N // ZR
FH = B * H // 2

def _sc_deg_body(dst_h, zeros_h, ones_h, out, dstv, onesb, zbuf, acc,
                 ds0, ds1, ds2, ds3):
    dsems = [ds0, ds1, ds2, ds3]
    c = lax.axis_index("c")
    s = lax.axis_index("s")
    w = c * NS + s
    pltpu.sync_copy(zeros_h, zbuf)
    pltpu.sync_copy(ones_h, onesb)
    for t in range((NZC + NS - 1) // NS):
        q = s + t * NS

        @pl.when(q < NZC)
        def _():
            pltpu.sync_copy(zbuf, acc.at[pl.ds(q * ZR, ZR)])

    plsc.subcore_barrier()

    def block(b, carry):
        pltpu.sync_copy(dst_h.at[w, b], dstv)
        sd = [None] * BID
        for j in range(BID):
            if j >= 4:
                sd[j - 4].wait()
            sd[j] = pltpu.async_copy(onesb, acc.at[dstv.at[j]], dsems[j % 4],
                                     add=True)
        for j in range(BID - 4, BID):
            sd[j].wait()
        return carry

    lax.fori_loop(0, NBID, block, 0)
    plsc.subcore_barrier()
    for t in range((NZC + NS - 1) // NS):
        q = s + t * NS

        @pl.when(q < NZC)
        def _():
            pltpu.sync_copy(acc.at[pl.ds(q * ZR, ZR)], zbuf)
            pltpu.sync_copy(zbuf, out.at[c * NZC + q])


DEPTH = 4


def _sc_agg_body(xs0, xs1, src_h, dst_h, zeros_h, out, srcv, dstv, r0, r1,
                 r2, r3, zbuf, acc, g0, g1, g2, g3, t0, t1, t2, t3):
    rows = [r0, r1, r2, r3]
    semg = [g0, g1, g2, g3]
    sems = [t0, t1, t2, t3]
    c = lax.axis_index("c")
    s = lax.axis_index("s")
    pltpu.sync_copy(zeros_h, zbuf)
    for t in range((NZC + NS - 1) // NS):
        q = s + t * NS

        @pl.when(q < NZC)
        def _():
            pltpu.sync_copy(zbuf, acc.at[pl.ds(q * ZR, ZR)])

    plsc.subcore_barrier()

    def block(b, carry):
        pltpu.sync_copy(src_h.at[s, b], srcv)
        pltpu.sync_copy(dst_h.at[s, b], dstv)

        def run(xs):
            gd = [None] * BI
            sd = [None] * BI
            for j in range(BI):
                sl = j % DEPTH
                if j >= DEPTH:
                    sd[j - DEPTH].wait()
                gd[j] = pltpu.async_copy(xs.at[srcv.at[j]], rows[sl],
                                         semg[sl])
                if j >= DEPTH - 1:
                    k2 = j - (DEPTH - 1)
                    gd[k2].wait()
                    sd[k2] = pltpu.async_copy(
                        rows[k2 % DEPTH], acc.at[dstv.at[k2]],
                        sems[k2 % DEPTH], add=True)
            for j in range(BI - (DEPTH - 1), BI):
                gd[j].wait()
                sd[j] = pltpu.async_copy(rows[j % DEPTH],
                                         acc.at[dstv.at[j]],
                                         sems[j % DEPTH], add=True)
            for j in range(BI - DEPTH, BI):
                sd[j].wait()

        @pl.when(c == 0)
        def _():
            run(xs0)

        @pl.when(c == 1)
        def _():
            run(xs1)

        return carry

    lax.fori_loop(0, NBI, block, 0)
    plsc.subcore_barrier()
    for t in range((NZC + NS - 1) // NS):
        q = s + t * NS

        @pl.when(q < NZC)
        def _():
            pltpu.sync_copy(acc.at[pl.ds(q * ZR, ZR)], zbuf)
            pltpu.sync_copy(zbuf, out.at[c * NZC + q])


@functools.cache
def _sc_kernels():
    mesh = plsc.VectorSubcoreMesh(
        core_axis_name="c", subcore_axis_name="s",
        num_cores=NC, num_subcores=NS)
    sc_deg = pl.kernel(
        _sc_deg_body,
        out_type=jax.ShapeDtypeStruct((NC * NZC, ZR, FH), jnp.float32),
        mesh=mesh,
        scratch_types=[
            pltpu.VMEM((BID, K), jnp.int32),
            pltpu.VMEM((K, FH), jnp.float32),
            pltpu.VMEM((ZR, FH), jnp.float32),
            pltpu.VMEM_SHARED((N, FH), jnp.float32),
        ] + [pltpu.SemaphoreType.DMA] * 4,
    )
    sc_agg = pl.kernel(
        _sc_agg_body,
        out_type=jax.ShapeDtypeStruct((NC * NZC, ZR, FH), jnp.float32),
        mesh=mesh,
        scratch_types=[
            pltpu.VMEM((BI, K), jnp.int32),
            pltpu.VMEM((BI, K), jnp.int32),
            pltpu.VMEM((K, FH), jnp.float32),
            pltpu.VMEM((K, FH), jnp.float32),
            pltpu.VMEM((K, FH), jnp.float32),
            pltpu.VMEM((K, FH), jnp.float32),
            pltpu.VMEM((ZR, FH), jnp.float32),
            pltpu.VMEM_SHARED((N, FH), jnp.float32),
        ] + [pltpu.SemaphoreType.DMA] * 8,
    )
    return sc_deg, sc_agg


BN = 1000
NBLK = N // BN


def _tc_prep_body(x_ref, w_ref, dega_ref, degb_ref, xs0_ref, xs1_ref,
                  dinv_ref):
    deg = dega_ref[:, 0] + degb_ref[:, 0] + 1.0
    dinv = lax.rsqrt(deg)[:, None]
    w = w_ref[...]
    xw = [jnp.dot(x_ref[b], w, preferred_element_type=jnp.float32) * dinv
          for b in range(B)]
    xs0_ref[...] = jnp.concatenate([xw[0], xw[1]], axis=1)
    xs1_ref[...] = jnp.concatenate([xw[2], xw[3]], axis=1)
    dinv_ref[...] = dinv


def _tc_prep(x, w1t, degs):
    return pl.pallas_call(
        _tc_prep_body,
        grid=(NBLK,),
        in_specs=[
            pl.BlockSpec((B, BN, F_IN), lambda i: (0, i, 0)),
            pl.BlockSpec((F_IN, H), lambda i: (0, 0)),
            pl.BlockSpec((BN, FH), lambda i: (i, 0)),
            pl.BlockSpec((BN, FH), lambda i: (NBLK + i, 0)),
        ],
        out_specs=[
            pl.BlockSpec((BN, FH), lambda i: (i, 0)),
            pl.BlockSpec((BN, FH), lambda i: (i, 0)),
            pl.BlockSpec((BN, 1), lambda i: (i, 0)),
        ],
        out_shape=[
            jax.ShapeDtypeStruct((N, FH), jnp.float32),
            jax.ShapeDtypeStruct((N, FH), jnp.float32),
            jax.ShapeDtypeStruct((N, 1), jnp.float32),
        ],
    )(x, w1t, degs, degs)


def _ln_relu(seg, g, be):
    mu = jnp.mean(seg, axis=1, keepdims=True)
    d = seg - mu
    var = jnp.mean(d * d, axis=1, keepdims=True)
    return jnp.maximum(d * lax.rsqrt(var + 1e-5) * g + be, 0.0)


def _tc_mid_body(agga_ref, aggb_ref, xs0_ref, xs1_ref, dinv_ref, b1_ref,
                 g1_ref, be1_ref, w2_ref, o0_ref, o1_ref):
    dinv = dinv_ref[...]
    b1, g1, be1 = b1_ref[...], g1_ref[...], be1_ref[...]
    w2 = w2_ref[...]
    for agg_ref, xs_ref, o_ref in [(agga_ref, xs0_ref, o0_ref),
                                   (aggb_ref, xs1_ref, o1_ref)]:
        outs = []
        for j in range(2):
            sl = slice(j * H, (j + 1) * H)
            pre = dinv * (agg_ref[:, sl] + xs_ref[:, sl]) + b1
            z = _ln_relu(pre, g1, be1)
            outs.append(
                jnp.dot(z, w2, preferred_element_type=jnp.float32) * dinv)
        o_ref[...] = jnp.concatenate(outs, axis=1)


def _tc_mid(agg1, xs0, xs1, dinv, b1, g1, be1, w2t):
    return pl.pallas_call(
        _tc_mid_body,
        grid=(NBLK,),
        in_specs=[
            pl.BlockSpec((BN, FH), lambda i: (i, 0)),
            pl.BlockSpec((BN, FH), lambda i: (NBLK + i, 0)),
            pl.BlockSpec((BN, FH), lambda i: (i, 0)),
            pl.BlockSpec((BN, FH), lambda i: (i, 0)),
            pl.BlockSpec((BN, 1), lambda i: (i, 0)),
            pl.BlockSpec((1, H), lambda i: (0, 0)),
            pl.BlockSpec((1, H), lambda i: (0, 0)),
            pl.BlockSpec((1, H), lambda i: (0, 0)),
            pl.BlockSpec((H, D_OUT), lambda i: (0, 0)),
        ],
        out_specs=[
            pl.BlockSpec((BN, FH), lambda i: (i, 0)),
            pl.BlockSpec((BN, FH), lambda i: (i, 0)),
        ],
        out_shape=[
            jax.ShapeDtypeStruct((N, FH), jnp.float32),
            jax.ShapeDtypeStruct((N, FH), jnp.float32),
        ],
    )(agg1, agg1, xs0, xs1, dinv, b1, g1, be1, w2t)


def _tc_post_body(agga_ref, aggb_ref, xs0_ref, xs1_ref, dinv_ref, b2_ref,
                  g2_ref, be2_ref, embs_ref, graph_ref):
    i = pl.program_id(0)
    dinv = dinv_ref[...]
    b2, g2, be2 = b2_ref[...], g2_ref[...], be2_ref[...]
    ys = []
    for agg_ref, xs_ref in [(agga_ref, xs0_ref), (aggb_ref, xs1_ref)]:
        for j in range(2):
            sl = slice(j * D_OUT, (j + 1) * D_OUT)
            pre = dinv * (agg_ref[:, sl] + xs_ref[:, sl]) + b2
            ys.append(_ln_relu(pre, g2, be2))
    stacked = jnp.stack(ys, axis=0)
    embs_ref[...] = stacked
    part = jnp.sum(stacked, axis=1)

    @pl.when(i == 0)
    def _():
        graph_ref[...] = part

    @pl.when(i > 0)
    def _():
        graph_ref[...] = graph_ref[...] + part

    @pl.when(i == NBLK - 1)
    def _():
        graph_ref[...] = graph_ref[...] * (1.0 / N)


def _tc_post(agg2, xs2_0, xs2_1, dinv, b2, g2, be2):
    return pl.pallas_call(
        _tc_post_body,
        grid=(NBLK,),
        in_specs=[
            pl.BlockSpec((BN, FH), lambda i: (i, 0)),
            pl.BlockSpec((BN, FH), lambda i: (NBLK + i, 0)),
            pl.BlockSpec((BN, FH), lambda i: (i, 0)),
            pl.BlockSpec((BN, FH), lambda i: (i, 0)),
            pl.BlockSpec((BN, 1), lambda i: (i, 0)),
            pl.BlockSpec((1, D_OUT), lambda i: (0, 0)),
            pl.BlockSpec((1, D_OUT), lambda i: (0, 0)),
            pl.BlockSpec((1, D_OUT), lambda i: (0, 0)),
        ],
        out_specs=[
            pl.BlockSpec((B, BN, D_OUT), lambda i: (0, i, 0)),
            pl.BlockSpec((B, D_OUT), lambda i: (0, 0)),
        ],
        out_shape=[
            jax.ShapeDtypeStruct((B, N, D_OUT), jnp.float32),
            jax.ShapeDtypeStruct((B, D_OUT), jnp.float32),
        ],
    )(agg2, agg2, xs2_0, xs2_1, dinv, b2, g2, be2)


def kernel(node_features, edge_index, W1, b1, g1, be1, W2, b2, g2, be2):
    src_a = edge_index[0].reshape(NS, NBI, BI, K)
    dst_a = edge_index[1].reshape(NS, NBI, BI, K)
    dst_d = edge_index[1].reshape(NW, NBID, BID, K)
    zeros_fh = jnp.zeros((ZR, FH), jnp.float32)
    ones_fh = jnp.ones((K, FH), jnp.float32)

    sc_deg, sc_agg = _sc_kernels()
    degs = sc_deg(dst_d, zeros_fh, ones_fh).reshape(NC * N, FH)
    xs0, xs1, dinv = _tc_prep(node_features, W1.T, degs)
    agg1 = sc_agg(xs0, xs1, src_a, dst_a, zeros_fh).reshape(NC * N, FH)
    xs2_0, xs2_1 = _tc_mid(agg1, xs0, xs1, dinv,
                           b1.reshape(1, H), g1.reshape(1, H),
                           be1.reshape(1, H), W2.T)
    agg2 = sc_agg(xs2_0, xs2_1, src_a, dst_a,
                  zeros_fh).reshape(NC * N, FH)
    node_embs, graph_emb = _tc_post(agg2, xs2_0, xs2_1, dinv,
                                    b2.reshape(1, D_OUT),
                                    g2.reshape(1, D_OUT),
                                    be2.reshape(1, D_OUT))
    return (node_embs, graph_emb)


# speedup vs baseline: 50.1881x; 1.0439x over previous
import functools

import jax
import jax.numpy as jnp
from jax import lax
from jax.experimental import pallas as pl
from jax.experimental.pallas import tpu as pltpu
from jax.experimental.pallas import tpu_sc as plsc

B, N, F_IN, H, D_OUT, E = 4, 10000, 128, 64, 64, 320000
NC, NS = 2, 16
NW = NC * NS
K = 80
NCHD = (E // NW) // K
NCHA = (E // NS) // K
KA = 128
EPT = E // NS
BI = 12
NBI = 13
TAIL = EPT - NBI * BI * KA
BID = 25
NBID = NCHD // BID
ZR = 40
NZC = N // ZR
FH = B * H // 2

def _sc_deg_body(dst_h, zeros_h, ones_h, out, dstv, onesb, zbuf, acc,
                 ds0, ds1, ds2, ds3):
    dsems = [ds0, ds1, ds2, ds3]
    c = lax.axis_index("c")
    s = lax.axis_index("s")
    w = c * NS + s
    pltpu.sync_copy(zeros_h, zbuf)
    pltpu.sync_copy(ones_h, onesb)
    for t in range((NZC + NS - 1) // NS):
        q = s + t * NS

        @pl.when(q < NZC)
        def _():
            pltpu.sync_copy(zbuf, acc.at[pl.ds(q * ZR, ZR)])

    plsc.subcore_barrier()

    def block(b, carry):
        pltpu.sync_copy(dst_h.at[w, b], dstv)
        sd = [None] * BID
        for j in range(BID):
            if j >= 4:
                sd[j - 4].wait()
            sd[j] = pltpu.async_copy(onesb, acc.at[dstv.at[j]], dsems[j % 4],
                                     add=True)
        for j in range(BID - 4, BID):
            sd[j].wait()
        return carry

    lax.fori_loop(0, NBID, block, 0)
    plsc.subcore_barrier()
    for t in range((NZC + NS - 1) // NS):
        q = s + t * NS

        @pl.when(q < NZC)
        def _():
            pltpu.sync_copy(acc.at[pl.ds(q * ZR, ZR)], zbuf)
            pltpu.sync_copy(zbuf, out.at[c * NZC + q])


DEPTH = 2


def _sc_agg_body(xs0, xs1, src_h, dst_h, srct_h, dstt_h, zeros_h, out,
                 srcv, dstv, stv, dtv, r0, r1, zbuf, acc, g0, g1, t0, t1):
    rows = [r0, r1]
    semg = [g0, g1]
    sems = [t0, t1]
    c = lax.axis_index("c")
    s = lax.axis_index("s")
    pltpu.sync_copy(zeros_h, zbuf)
    for t in range((NZC + NS - 1) // NS):
        q = s + t * NS

        @pl.when(q < NZC)
        def _():
            pltpu.sync_copy(zbuf, acc.at[pl.ds(q * ZR, ZR)])

    plsc.subcore_barrier()

    def block(b, carry):
        pltpu.sync_copy(src_h.at[s, b], srcv)
        pltpu.sync_copy(dst_h.at[s, b], dstv)

        def run(xs):
            gd = [None] * BI
            sd = [None] * BI
            for j in range(BI):
                sl = j % DEPTH
                if j >= DEPTH:
                    sd[j - DEPTH].wait()
                gd[j] = pltpu.async_copy(xs.at[srcv.at[j]], rows[sl],
                                         semg[sl])
                if j >= DEPTH - 1:
                    k2 = j - (DEPTH - 1)
                    gd[k2].wait()
                    sd[k2] = pltpu.async_copy(
                        rows[k2 % DEPTH], acc.at[dstv.at[k2]],
                        sems[k2 % DEPTH], add=True)
            for j in range(BI - (DEPTH - 1), BI):
                gd[j].wait()
                sd[j] = pltpu.async_copy(rows[j % DEPTH],
                                         acc.at[dstv.at[j]],
                                         sems[j % DEPTH], add=True)
            for j in range(BI - DEPTH, BI):
                sd[j].wait()

        @pl.when(c == 0)
        def _():
            run(xs0)

        @pl.when(c == 1)
        def _():
            run(xs1)

        return carry

    lax.fori_loop(0, NBI, block, 0)

    pltpu.sync_copy(srct_h.at[s], stv)
    pltpu.sync_copy(dstt_h.at[s], dtv)
    tr = rows[0].at[pl.ds(0, TAIL)]

    def run_tail(xs):
        pltpu.async_copy(xs.at[stv.at[0]], tr, semg[0]).wait()
        pltpu.async_copy(tr, acc.at[dtv.at[0]], sems[0], add=True).wait()

    @pl.when(c == 0)
    def _():
        run_tail(xs0)

    @pl.when(c == 1)
    def _():
        run_tail(xs1)

    plsc.subcore_barrier()
    for t in range((NZC + NS - 1) // NS):
        q = s + t * NS

        @pl.when(q < NZC)
        def _():
            pltpu.sync_copy(acc.at[pl.ds(q * ZR, ZR)], zbuf)
            pltpu.sync_copy(zbuf, out.at[c * NZC + q])


@functools.cache
def _sc_kernels():
    mesh = plsc.VectorSubcoreMesh(
        core_axis_name="c", subcore_axis_name="s",
        num_cores=NC, num_subcores=NS)
    sc_deg = pl.kernel(
        _sc_deg_body,
        out_type=jax.ShapeDtypeStruct((NC * NZC, ZR, FH), jnp.float32),
        mesh=mesh,
        scratch_types=[
            pltpu.VMEM((BID, K), jnp.int32),
            pltpu.VMEM((K, FH), jnp.float32),
            pltpu.VMEM((ZR, FH), jnp.float32),
            pltpu.VMEM_SHARED((N, FH), jnp.float32),
        ] + [pltpu.SemaphoreType.DMA] * 4,
    )
    sc_agg = pl.kernel(
        _sc_agg_body,
        out_type=jax.ShapeDtypeStruct((NC * NZC, ZR, FH), jnp.float32),
        mesh=mesh,
        scratch_types=[
            pltpu.VMEM((BI, KA), jnp.int32),
            pltpu.VMEM((BI, KA), jnp.int32),
            pltpu.VMEM((1, TAIL), jnp.int32),
            pltpu.VMEM((1, TAIL), jnp.int32),
            pltpu.VMEM((KA, FH), jnp.float32),
            pltpu.VMEM((KA, FH), jnp.float32),
            pltpu.VMEM((ZR, FH), jnp.float32),
            pltpu.VMEM_SHARED((N, FH), jnp.float32),
        ] + [pltpu.SemaphoreType.DMA] * 4,
    )
    return sc_deg, sc_agg


BN = 1000
NBLK = N // BN


def _tc_prep_body(x_ref, w_ref, dega_ref, degb_ref, xs0_ref, xs1_ref,
                  dinv_ref):
    deg = dega_ref[:, 0] + degb_ref[:, 0] + 1.0
    dinv = lax.rsqrt(deg)[:, None]
    w = w_ref[...]
    xw = [jnp.dot(x_ref[b], w, preferred_element_type=jnp.float32) * dinv
          for b in range(B)]
    xs0_ref[...] = jnp.concatenate([xw[0], xw[1]], axis=1)
    xs1_ref[...] = jnp.concatenate([xw[2], xw[3]], axis=1)
    dinv_ref[...] = dinv


def _tc_prep(x, w1t, degs):
    return pl.pallas_call(
        _tc_prep_body,
        grid=(NBLK,),
        in_specs=[
            pl.BlockSpec((B, BN, F_IN), lambda i: (0, i, 0)),
            pl.BlockSpec((F_IN, H), lambda i: (0, 0)),
            pl.BlockSpec((BN, FH), lambda i: (i, 0)),
            pl.BlockSpec((BN, FH), lambda i: (NBLK + i, 0)),
        ],
        out_specs=[
            pl.BlockSpec((BN, FH), lambda i: (i, 0)),
            pl.BlockSpec((BN, FH), lambda i: (i, 0)),
            pl.BlockSpec((BN, 1), lambda i: (i, 0)),
        ],
        out_shape=[
            jax.ShapeDtypeStruct((N, FH), jnp.float32),
            jax.ShapeDtypeStruct((N, FH), jnp.float32),
            jax.ShapeDtypeStruct((N, 1), jnp.float32),
        ],
    )(x, w1t, degs, degs)


def _ln_relu(seg, g, be):
    mu = jnp.mean(seg, axis=1, keepdims=True)
    d = seg - mu
    var = jnp.mean(d * d, axis=1, keepdims=True)
    return jnp.maximum(d * lax.rsqrt(var + 1e-5) * g + be, 0.0)


def _tc_mid_body(agga_ref, aggb_ref, xs0_ref, xs1_ref, dinv_ref, b1_ref,
                 g1_ref, be1_ref, w2_ref, o0_ref, o1_ref):
    dinv = dinv_ref[...]
    b1, g1, be1 = b1_ref[...], g1_ref[...], be1_ref[...]
    w2 = w2_ref[...]
    for agg_ref, xs_ref, o_ref in [(agga_ref, xs0_ref, o0_ref),
                                   (aggb_ref, xs1_ref, o1_ref)]:
        outs = []
        for j in range(2):
            sl = slice(j * H, (j + 1) * H)
            pre = dinv * (agg_ref[:, sl] + xs_ref[:, sl]) + b1
            z = _ln_relu(pre, g1, be1)
            outs.append(
                jnp.dot(z, w2, preferred_element_type=jnp.float32) * dinv)
        o_ref[...] = jnp.concatenate(outs, axis=1)


def _tc_mid(agg1, xs0, xs1, dinv, b1, g1, be1, w2t):
    return pl.pallas_call(
        _tc_mid_body,
        grid=(NBLK,),
        in_specs=[
            pl.BlockSpec((BN, FH), lambda i: (i, 0)),
            pl.BlockSpec((BN, FH), lambda i: (NBLK + i, 0)),
            pl.BlockSpec((BN, FH), lambda i: (i, 0)),
            pl.BlockSpec((BN, FH), lambda i: (i, 0)),
            pl.BlockSpec((BN, 1), lambda i: (i, 0)),
            pl.BlockSpec((1, H), lambda i: (0, 0)),
            pl.BlockSpec((1, H), lambda i: (0, 0)),
            pl.BlockSpec((1, H), lambda i: (0, 0)),
            pl.BlockSpec((H, D_OUT), lambda i: (0, 0)),
        ],
        out_specs=[
            pl.BlockSpec((BN, FH), lambda i: (i, 0)),
            pl.BlockSpec((BN, FH), lambda i: (i, 0)),
        ],
        out_shape=[
            jax.ShapeDtypeStruct((N, FH), jnp.float32),
            jax.ShapeDtypeStruct((N, FH), jnp.float32),
        ],
    )(agg1, agg1, xs0, xs1, dinv, b1, g1, be1, w2t)


def _tc_post_body(agga_ref, aggb_ref, xs0_ref, xs1_ref, dinv_ref, b2_ref,
                  g2_ref, be2_ref, embs_ref, graph_ref):
    i = pl.program_id(0)
    dinv = dinv_ref[...]
    b2, g2, be2 = b2_ref[...], g2_ref[...], be2_ref[...]
    ys = []
    for agg_ref, xs_ref in [(agga_ref, xs0_ref), (aggb_ref, xs1_ref)]:
        for j in range(2):
            sl = slice(j * D_OUT, (j + 1) * D_OUT)
            pre = dinv * (agg_ref[:, sl] + xs_ref[:, sl]) + b2
            ys.append(_ln_relu(pre, g2, be2))
    stacked = jnp.stack(ys, axis=0)
    embs_ref[...] = stacked
    part = jnp.sum(stacked, axis=1)

    @pl.when(i == 0)
    def _():
        graph_ref[...] = part

    @pl.when(i > 0)
    def _():
        graph_ref[...] = graph_ref[...] + part

    @pl.when(i == NBLK - 1)
    def _():
        graph_ref[...] = graph_ref[...] * (1.0 / N)


def _tc_post(agg2, xs2_0, xs2_1, dinv, b2, g2, be2):
    return pl.pallas_call(
        _tc_post_body,
        grid=(NBLK,),
        in_specs=[
            pl.BlockSpec((BN, FH), lambda i: (i, 0)),
            pl.BlockSpec((BN, FH), lambda i: (NBLK + i, 0)),
            pl.BlockSpec((BN, FH), lambda i: (i, 0)),
            pl.BlockSpec((BN, FH), lambda i: (i, 0)),
            pl.BlockSpec((BN, 1), lambda i: (i, 0)),
            pl.BlockSpec((1, D_OUT), lambda i: (0, 0)),
            pl.BlockSpec((1, D_OUT), lambda i: (0, 0)),
            pl.BlockSpec((1, D_OUT), lambda i: (0, 0)),
        ],
        out_specs=[
            pl.BlockSpec((B, BN, D_OUT), lambda i: (0, i, 0)),
            pl.BlockSpec((B, D_OUT), lambda i: (0, 0)),
        ],
        out_shape=[
            jax.ShapeDtypeStruct((B, N, D_OUT), jnp.float32),
            jax.ShapeDtypeStruct((B, D_OUT), jnp.float32),
        ],
    )(agg2, agg2, xs2_0, xs2_1, dinv, b2, g2, be2)


def kernel(node_features, edge_index, W1, b1, g1, be1, W2, b2, g2, be2):
    src_t = edge_index[0].reshape(NS, EPT)
    dst_t = edge_index[1].reshape(NS, EPT)
    src_a = src_t[:, :NBI * BI * KA].reshape(NS, NBI, BI, KA)
    dst_a = dst_t[:, :NBI * BI * KA].reshape(NS, NBI, BI, KA)
    src_tl = src_t[:, NBI * BI * KA:].reshape(NS, 1, TAIL)
    dst_tl = dst_t[:, NBI * BI * KA:].reshape(NS, 1, TAIL)
    dst_d = edge_index[1].reshape(NW, NBID, BID, K)
    zeros_fh = jnp.zeros((ZR, FH), jnp.float32)
    ones_fh = jnp.ones((K, FH), jnp.float32)

    sc_deg, sc_agg = _sc_kernels()
    degs = sc_deg(dst_d, zeros_fh, ones_fh).reshape(NC * N, FH)
    xs0, xs1, dinv = _tc_prep(node_features, W1.T, degs)
    agg1 = sc_agg(xs0, xs1, src_a, dst_a, src_tl, dst_tl,
                  zeros_fh).reshape(NC * N, FH)
    xs2_0, xs2_1 = _tc_mid(agg1, xs0, xs1, dinv,
                           b1.reshape(1, H), g1.reshape(1, H),
                           be1.reshape(1, H), W2.T)
    agg2 = sc_agg(xs2_0, xs2_1, src_a, dst_a, src_tl, dst_tl,
                  zeros_fh).reshape(NC * N, FH)
    node_embs, graph_emb = _tc_post(agg2, xs2_0, xs2_1, dinv,
                                    b2.reshape(1, D_OUT),
                                    g2.reshape(1, D_OUT),
                                    be2.reshape(1, D_OUT))
    return (node_embs, graph_emb)


# speedup vs baseline: 52.7353x; 1.0508x over previous
import functools

import jax
import jax.numpy as jnp
from jax import lax
from jax.experimental import pallas as pl
from jax.experimental.pallas import tpu as pltpu
from jax.experimental.pallas import tpu_sc as plsc

B, N, F_IN, H, D_OUT, E = 4, 10000, 128, 64, 64, 320000
NC, NS = 2, 16
NW = NC * NS
K = 80
NCHD = (E // NW) // K
NCHA = (E // NS) // K
KA = 128
EPT = E // NS
BI = 26
NBI = 6
TAIL = EPT - NBI * BI * KA
BID = 25
NBID = NCHD // BID
ZR = 40
NZC = N // ZR
FH = B * H // 2

def _sc_deg_body(dst_h, zeros_h, ones_h, out, dstv, onesb, zbuf, acc,
                 ds0, ds1, ds2, ds3):
    dsems = [ds0, ds1, ds2, ds3]
    c = lax.axis_index("c")
    s = lax.axis_index("s")
    w = c * NS + s
    pltpu.sync_copy(zeros_h, zbuf)
    pltpu.sync_copy(ones_h, onesb)
    for t in range((NZC + NS - 1) // NS):
        q = s + t * NS

        @pl.when(q < NZC)
        def _():
            pltpu.sync_copy(zbuf, acc.at[pl.ds(q * ZR, ZR)])

    plsc.subcore_barrier()

    def block(b, carry):
        pltpu.sync_copy(dst_h.at[w, b], dstv)
        sd = [None] * BID
        for j in range(BID):
            if j >= 4:
                sd[j - 4].wait()
            sd[j] = pltpu.async_copy(onesb, acc.at[dstv.at[j]], dsems[j % 4],
                                     add=True)
        for j in range(BID - 4, BID):
            sd[j].wait()
        return carry

    lax.fori_loop(0, NBID, block, 0)
    plsc.subcore_barrier()
    for t in range((NZC + NS - 1) // NS):
        q = s + t * NS

        @pl.when(q < NZC)
        def _():
            pltpu.sync_copy(acc.at[pl.ds(q * ZR, ZR)], zbuf)
            pltpu.sync_copy(zbuf, out.at[c * NZC + q])


DEPTH = 2


def _sc_agg_body(xs0, xs1, src_h, dst_h, srct_h, dstt_h, zeros_h, out,
                 srcv, dstv, stv, dtv, r0, r1, zbuf, acc, g0, g1, t0, t1):
    rows = [r0, r1]
    semg = [g0, g1]
    sems = [t0, t1]
    c = lax.axis_index("c")
    s = lax.axis_index("s")
    pltpu.sync_copy(zeros_h, zbuf)
    for t in range((NZC + NS - 1) // NS):
        q = s + t * NS

        @pl.when(q < NZC)
        def _():
            pltpu.sync_copy(zbuf, acc.at[pl.ds(q * ZR, ZR)])

    plsc.subcore_barrier()

    def block(b, carry):
        pltpu.sync_copy(src_h.at[s, b], srcv)
        pltpu.sync_copy(dst_h.at[s, b], dstv)

        def run(xs):
            gd = [None] * BI
            sd = [None] * BI
            for j in range(BI):
                sl = j % DEPTH
                if j >= DEPTH:
                    sd[j - DEPTH].wait()
                gd[j] = pltpu.async_copy(xs.at[srcv.at[j]], rows[sl],
                                         semg[sl])
                if j >= DEPTH - 1:
                    k2 = j - (DEPTH - 1)
                    gd[k2].wait()
                    sd[k2] = pltpu.async_copy(
                        rows[k2 % DEPTH], acc.at[dstv.at[k2]],
                        sems[k2 % DEPTH], add=True)
            for j in range(BI - (DEPTH - 1), BI):
                gd[j].wait()
                sd[j] = pltpu.async_copy(rows[j % DEPTH],
                                         acc.at[dstv.at[j]],
                                         sems[j % DEPTH], add=True)
            for j in range(BI - DEPTH, BI):
                sd[j].wait()

        @pl.when(c == 0)
        def _():
            run(xs0)

        @pl.when(c == 1)
        def _():
            run(xs1)

        return carry

    lax.fori_loop(0, NBI, block, 0)

    pltpu.sync_copy(srct_h.at[s], stv)
    pltpu.sync_copy(dstt_h.at[s], dtv)
    tr = rows[0].at[pl.ds(0, TAIL)]

    def run_tail(xs):
        pltpu.async_copy(xs.at[stv.at[0]], tr, semg[0]).wait()
        pltpu.async_copy(tr, acc.at[dtv.at[0]], sems[0], add=True).wait()

    @pl.when(c == 0)
    def _():
        run_tail(xs0)

    @pl.when(c == 1)
    def _():
        run_tail(xs1)

    plsc.subcore_barrier()
    for t in range((NZC + NS - 1) // NS):
        q = s + t * NS

        @pl.when(q < NZC)
        def _():
            pltpu.sync_copy(acc.at[pl.ds(q * ZR, ZR)], zbuf)
            pltpu.sync_copy(zbuf, out.at[c * NZC + q])


@functools.cache
def _sc_kernels():
    mesh = plsc.VectorSubcoreMesh(
        core_axis_name="c", subcore_axis_name="s",
        num_cores=NC, num_subcores=NS)
    sc_deg = pl.kernel(
        _sc_deg_body,
        out_type=jax.ShapeDtypeStruct((NC * NZC, ZR, FH), jnp.float32),
        mesh=mesh,
        scratch_types=[
            pltpu.VMEM((BID, K), jnp.int32),
            pltpu.VMEM((K, FH), jnp.float32),
            pltpu.VMEM((ZR, FH), jnp.float32),
            pltpu.VMEM_SHARED((N, FH), jnp.float32),
        ] + [pltpu.SemaphoreType.DMA] * 4,
    )
    sc_agg = pl.kernel(
        _sc_agg_body,
        out_type=jax.ShapeDtypeStruct((NC * NZC, ZR, FH), jnp.float32),
        mesh=mesh,
        scratch_types=[
            pltpu.VMEM((BI, KA), jnp.int32),
            pltpu.VMEM((BI, KA), jnp.int32),
            pltpu.VMEM((1, TAIL), jnp.int32),
            pltpu.VMEM((1, TAIL), jnp.int32),
            pltpu.VMEM((KA, FH), jnp.float32),
            pltpu.VMEM((KA, FH), jnp.float32),
            pltpu.VMEM((ZR, FH), jnp.float32),
            pltpu.VMEM_SHARED((N, FH), jnp.float32),
        ] + [pltpu.SemaphoreType.DMA] * 4,
    )
    return sc_deg, sc_agg


BN = 1000
NBLK = N // BN


def _tc_prep_body(x_ref, w_ref, dega_ref, degb_ref, xs0_ref, xs1_ref,
                  dinv_ref):
    deg = dega_ref[:, 0] + degb_ref[:, 0] + 1.0
    dinv = lax.rsqrt(deg)[:, None]
    w = w_ref[...]
    xw = [jnp.dot(x_ref[b], w, preferred_element_type=jnp.float32) * dinv
          for b in range(B)]
    xs0_ref[...] = jnp.concatenate([xw[0], xw[1]], axis=1)
    xs1_ref[...] = jnp.concatenate([xw[2], xw[3]], axis=1)
    dinv_ref[...] = dinv


def _tc_prep(x, w1t, degs):
    return pl.pallas_call(
        _tc_prep_body,
        grid=(NBLK,),
        in_specs=[
            pl.BlockSpec((B, BN, F_IN), lambda i: (0, i, 0)),
            pl.BlockSpec((F_IN, H), lambda i: (0, 0)),
            pl.BlockSpec((BN, FH), lambda i: (i, 0)),
            pl.BlockSpec((BN, FH), lambda i: (NBLK + i, 0)),
        ],
        out_specs=[
            pl.BlockSpec((BN, FH), lambda i: (i, 0)),
            pl.BlockSpec((BN, FH), lambda i: (i, 0)),
            pl.BlockSpec((BN, 1), lambda i: (i, 0)),
        ],
        out_shape=[
            jax.ShapeDtypeStruct((N, FH), jnp.float32),
            jax.ShapeDtypeStruct((N, FH), jnp.float32),
            jax.ShapeDtypeStruct((N, 1), jnp.float32),
        ],
    )(x, w1t, degs, degs)


def _ln_relu(seg, g, be):
    mu = jnp.mean(seg, axis=1, keepdims=True)
    d = seg - mu
    var = jnp.mean(d * d, axis=1, keepdims=True)
    return jnp.maximum(d * lax.rsqrt(var + 1e-5) * g + be, 0.0)


def _tc_mid_body(agga_ref, aggb_ref, xs0_ref, xs1_ref, dinv_ref, b1_ref,
                 g1_ref, be1_ref, w2_ref, o0_ref, o1_ref):
    dinv = dinv_ref[...]
    b1, g1, be1 = b1_ref[...], g1_ref[...], be1_ref[...]
    w2 = w2_ref[...]
    for agg_ref, xs_ref, o_ref in [(agga_ref, xs0_ref, o0_ref),
                                   (aggb_ref, xs1_ref, o1_ref)]:
        outs = []
        for j in range(2):
            sl = slice(j * H, (j + 1) * H)
            pre = dinv * (agg_ref[:, sl] + xs_ref[:, sl]) + b1
            z = _ln_relu(pre, g1, be1)
            outs.append(
                jnp.dot(z, w2, preferred_element_type=jnp.float32) * dinv)
        o_ref[...] = jnp.concatenate(outs, axis=1)


def _tc_mid(agg1, xs0, xs1, dinv, b1, g1, be1, w2t):
    return pl.pallas_call(
        _tc_mid_body,
        grid=(NBLK,),
        in_specs=[
            pl.BlockSpec((BN, FH), lambda i: (i, 0)),
            pl.BlockSpec((BN, FH), lambda i: (NBLK + i, 0)),
            pl.BlockSpec((BN, FH), lambda i: (i, 0)),
            pl.BlockSpec((BN, FH), lambda i: (i, 0)),
            pl.BlockSpec((BN, 1), lambda i: (i, 0)),
            pl.BlockSpec((1, H), lambda i: (0, 0)),
            pl.BlockSpec((1, H), lambda i: (0, 0)),
            pl.BlockSpec((1, H), lambda i: (0, 0)),
            pl.BlockSpec((H, D_OUT), lambda i: (0, 0)),
        ],
        out_specs=[
            pl.BlockSpec((BN, FH), lambda i: (i, 0)),
            pl.BlockSpec((BN, FH), lambda i: (i, 0)),
        ],
        out_shape=[
            jax.ShapeDtypeStruct((N, FH), jnp.float32),
            jax.ShapeDtypeStruct((N, FH), jnp.float32),
        ],
    )(agg1, agg1, xs0, xs1, dinv, b1, g1, be1, w2t)


def _tc_post_body(agga_ref, aggb_ref, xs0_ref, xs1_ref, dinv_ref, b2_ref,
                  g2_ref, be2_ref, embs_ref, graph_ref):
    i = pl.program_id(0)
    dinv = dinv_ref[...]
    b2, g2, be2 = b2_ref[...], g2_ref[...], be2_ref[...]
    ys = []
    for agg_ref, xs_ref in [(agga_ref, xs0_ref), (aggb_ref, xs1_ref)]:
        for j in range(2):
            sl = slice(j * D_OUT, (j + 1) * D_OUT)
            pre = dinv * (agg_ref[:, sl] + xs_ref[:, sl]) + b2
            ys.append(_ln_relu(pre, g2, be2))
    stacked = jnp.stack(ys, axis=0)
    embs_ref[...] = stacked
    part = jnp.sum(stacked, axis=1)

    @pl.when(i == 0)
    def _():
        graph_ref[...] = part

    @pl.when(i > 0)
    def _():
        graph_ref[...] = graph_ref[...] + part

    @pl.when(i == NBLK - 1)
    def _():
        graph_ref[...] = graph_ref[...] * (1.0 / N)


def _tc_post(agg2, xs2_0, xs2_1, dinv, b2, g2, be2):
    return pl.pallas_call(
        _tc_post_body,
        grid=(NBLK,),
        in_specs=[
            pl.BlockSpec((BN, FH), lambda i: (i, 0)),
            pl.BlockSpec((BN, FH), lambda i: (NBLK + i, 0)),
            pl.BlockSpec((BN, FH), lambda i: (i, 0)),
            pl.BlockSpec((BN, FH), lambda i: (i, 0)),
            pl.BlockSpec((BN, 1), lambda i: (i, 0)),
            pl.BlockSpec((1, D_OUT), lambda i: (0, 0)),
            pl.BlockSpec((1, D_OUT), lambda i: (0, 0)),
            pl.BlockSpec((1, D_OUT), lambda i: (0, 0)),
        ],
        out_specs=[
            pl.BlockSpec((B, BN, D_OUT), lambda i: (0, i, 0)),
            pl.BlockSpec((B, D_OUT), lambda i: (0, 0)),
        ],
        out_shape=[
            jax.ShapeDtypeStruct((B, N, D_OUT), jnp.float32),
            jax.ShapeDtypeStruct((B, D_OUT), jnp.float32),
        ],
    )(agg2, agg2, xs2_0, xs2_1, dinv, b2, g2, be2)


def kernel(node_features, edge_index, W1, b1, g1, be1, W2, b2, g2, be2):
    src_t = edge_index[0].reshape(NS, EPT)
    dst_t = edge_index[1].reshape(NS, EPT)
    src_a = src_t[:, :NBI * BI * KA].reshape(NS, NBI, BI, KA)
    dst_a = dst_t[:, :NBI * BI * KA].reshape(NS, NBI, BI, KA)
    src_tl = src_t[:, NBI * BI * KA:].reshape(NS, 1, TAIL)
    dst_tl = dst_t[:, NBI * BI * KA:].reshape(NS, 1, TAIL)
    dst_d = edge_index[1].reshape(NW, NBID, BID, K)
    zeros_fh = jnp.zeros((ZR, FH), jnp.float32)
    ones_fh = jnp.ones((K, FH), jnp.float32)

    sc_deg, sc_agg = _sc_kernels()
    degs = sc_deg(dst_d, zeros_fh, ones_fh).reshape(NC * N, FH)
    xs0, xs1, dinv = _tc_prep(node_features, W1.T, degs)
    agg1 = sc_agg(xs0, xs1, src_a, dst_a, src_tl, dst_tl,
                  zeros_fh).reshape(NC * N, FH)
    xs2_0, xs2_1 = _tc_mid(agg1, xs0, xs1, dinv,
                           b1.reshape(1, H), g1.reshape(1, H),
                           be1.reshape(1, H), W2.T)
    agg2 = sc_agg(xs2_0, xs2_1, src_a, dst_a, src_tl, dst_tl,
                  zeros_fh).reshape(NC * N, FH)
    node_embs, graph_emb = _tc_post(agg2, xs2_0, xs2_1, dinv,
                                    b2.reshape(1, D_OUT),
                                    g2.reshape(1, D_OUT),
                                    be2.reshape(1, D_OUT))
    return (node_embs, graph_emb)
